# baseline (speedup 1.0000x reference)
import functools

import jax
import jax.numpy as jnp
from jax import lax
from jax.experimental import pallas as pl
from jax.experimental.pallas import tpu as pltpu
from jax.experimental.pallas import tpu_sc as plsc

EMB = 64
BATCH = 16384
NB = 500001
NB_AL = 499968
TAIL = NB - NB_AL

_info = plsc.get_sparse_core_info()
_NC, _NS = _info.num_cores, _info.num_subcores
_NW = _NC * _NS
_R = 15744
_W = 256
_NCH = 62
_CAP = 512
_NPIECE = BATCH // _CAP
_LSZ = BATCH + 16
_LDUMP = _LSZ - 1
_SSZ = _CAP + 16
_SDUMP = _SSZ - 1
_RDUMP = _CAP
_OUTR = BATCH + 128


def _iota16():
    return lax.iota(jnp.int32, 16)


@functools.partial(
    pl.kernel,
    out_type=jax.ShapeDtypeStruct((_OUTR, 128), jnp.float32),
    mesh=plsc.VectorSubcoreMesh(core_axis_name="c", subcore_axis_name="s"),
    scratch_types=[
        pltpu.VMEM((1024,), jnp.int32),
        pltpu.VMEM((_LSZ,), jnp.int32),
        pltpu.VMEM((_LSZ,), jnp.int32),
        pltpu.VMEM((_SSZ,), jnp.int32),
        pltpu.VMEM((_SSZ,), jnp.int32),
        pltpu.VMEM((8, 8, _W), jnp.float32),
        pltpu.VMEM((_CAP + 8, 128), jnp.float32),
        pltpu.VMEM((1, 128), jnp.int32),
        pltpu.SemaphoreType.DMA,
    ],
    compiler_params=pltpu.CompilerParams(use_tc_tiling_on_sc=True,
                                         needs_layout_passes=False),
)
def _gather(table_hbm, tail_hbm, idx_hbm, out_hbm,
            idx_buf, i_list, p_list, lane_list, q_list,
            chunk_v, rows_v, p_stage, sem):
    wid = lax.axis_index("s") * _NC + lax.axis_index("c")
    lo = wid * _R
    hi = lo + _R

    def scan_piece(pp, cnt):
        pltpu.sync_copy(idx_hbm.at[pl.ds(pp * 1024, 1024)], idx_buf)

        def group(g, cnt):
            i_vec = idx_buf[pl.ds(g * 16, 16)]
            p_vec = pp * 1024 + g * 16 + _iota16()
            m = (i_vec >= lo) & (i_vec < hi)
            mi = m.astype(jnp.int32)
            pos = cnt + plsc.cumsum(mi) - 1
            tgt = jnp.where(m, pos, _LDUMP)
            plsc.store_scatter(i_list, [tgt], i_vec)
            plsc.store_scatter(p_list, [tgt], p_vec)
            return cnt + jnp.sum(mi)

        return lax.fori_loop(0, 64, group, cnt)

    cnt = lax.fori_loop(0, 16, scan_piece, jnp.int32(0))

    def process_chunk(piece, pc, off, width):
        def sub(gg, scnt):
            qpos = piece * _CAP + gg * 16
            i_vec = i_list[pl.ds(qpos, 16)]
            ql_vec = gg * 16 + _iota16()
            m = (ql_vec < pc) & (i_vec >= off) & (i_vec < off + width)
            mi = m.astype(jnp.int32)
            pos = scnt + plsc.cumsum(mi) - 1
            tgt = jnp.where(m, pos, _SDUMP)
            plsc.store_scatter(lane_list, [tgt], i_vec - off)
            plsc.store_scatter(q_list, [tgt], ql_vec)
            return scnt + jnp.sum(mi)

        scnt = lax.fori_loop(0, _CAP // 16, sub, jnp.int32(0))

        def gath(sg, carry):
            @pl.when(sg * 16 < scnt)
            def _():
                lane_vec = lane_list[pl.ds(sg * 16, 16)]
                q_vec = q_list[pl.ds(sg * 16, 16)]
                valid = (sg * 16 + _iota16()) < scnt
                lane_vec = jnp.where(valid, lane_vec, 0)
                q_vec = jnp.where(valid, q_vec, _RDUMP)
                for a in range(8):
                    a_vec = jnp.full((16,), a, jnp.int32)
                    for b in range(8):
                        v = plsc.load_gather(
                            chunk_v,
                            [a_vec, jnp.full((16,), b, jnp.int32), lane_vec])
                        plsc.store_scatter(
                            rows_v,
                            [q_vec, jnp.full((16,), a * 8 + b, jnp.int32)], v)

            return carry

        lax.fori_loop(0, _CAP // 16, gath, 0)

    def piece_body(piece, carry):
        pc = jnp.clip(cnt - piece * _CAP, 0, _CAP)

        @pl.when(pc > 0)
        def _():
            def chunk_body(c, carry2):
                off = jnp.minimum(lo + c * _W, NB_AL - _W)
                cps = [pltpu.async_copy(
                    table_hbm.at[a, pl.ds(0, 8), pl.ds(off, _W)],
                    chunk_v.at[a], sem) for a in range(8)]
                for cp in cps:
                    cp.wait()
                process_chunk(piece, pc, off, _W)
                return carry2

            lax.fori_loop(0, _NCH, chunk_body, 0)

            cps = [pltpu.async_copy(
                tail_hbm.at[a],
                chunk_v.at[a, pl.ds(0, 8), pl.ds(0, 128)],
                sem) for a in range(8)]
            for cp in cps:
                cp.wait()
            process_chunk(piece, pc, NB_AL, 128)

            def out_body(ob, carry3):
                @pl.when(ob * 128 < pc)
                def _o():
                    for gg8 in range(8):
                        qpos = ob * 128 + gg8 * 16
                        pv = p_list[pl.ds(piece * _CAP + qpos, 16)]
                        valid = (qpos + _iota16()) < pc
                        dummy = BATCH + gg8 * 16 + _iota16()
                        p_stage[0, pl.ds(gg8 * 16, 16)] = jnp.where(
                            valid, pv, dummy)
                    pltpu.async_copy(rows_v.at[pl.ds(ob * 128, 128)],
                                     out_hbm.at[p_stage.at[0]], sem).wait()

                return carry3

            lax.fori_loop(0, _CAP // 128, out_body, 0)

        return carry

    lax.fori_loop(0, _NPIECE, piece_body, 0)


def kernel(node_ids, table):
    idx = node_ids.astype(jnp.int32)
    table_t3 = table.T.reshape(8, 8, NB)
    tail = jnp.pad(table_t3[:, :, NB_AL:], ((0, 0), (0, 0), (0, 128 - TAIL)))
    out = _gather(table_t3, tail, idx)
    return out[:BATCH, :EMB]


# speedup vs baseline: 1.4642x; 1.4642x over previous
import functools

import jax
import jax.numpy as jnp
from jax import lax
from jax.experimental import pallas as pl
from jax.experimental.pallas import tpu as pltpu
from jax.experimental.pallas import tpu_sc as plsc

EMB = 64
BATCH = 16384
NB = 500001
NB_AL = 499968
TAIL = NB - NB_AL

_info = plsc.get_sparse_core_info()
_NC, _NS = _info.num_cores, _info.num_subcores
_NW = _NC * _NS
_R = 15744
_W = 128
_NCH = 123
_CAP = 576
_NPIECE = -(-BATCH // _CAP)
_NOB = -(-_CAP // 128)
_LSZ = BATCH + 16
_LDUMP = _LSZ - 1
_SSZ = _CAP + 16
_SDUMP = _SSZ - 1
_NROW = _NOB * 128 + 64
_RDUMP = _NOB * 128
_OUTR = BATCH + 128


def _iota16():
    return lax.iota(jnp.int32, 16)


@functools.partial(
    pl.kernel,
    out_type=jax.ShapeDtypeStruct((_OUTR, 128), jnp.float32),
    mesh=plsc.VectorSubcoreMesh(core_axis_name="c", subcore_axis_name="s"),
    scratch_types=[
        pltpu.VMEM((1024,), jnp.int32),
        pltpu.VMEM((_LSZ,), jnp.int32),
        pltpu.VMEM((_SSZ,), jnp.int32),
        pltpu.VMEM((_SSZ,), jnp.int32),
        pltpu.VMEM((2, 8, 8, _W), jnp.float32),
        pltpu.VMEM((_NROW, 128), jnp.float32),
        pltpu.VMEM((1, 128), jnp.int32),
        pltpu.SemaphoreType.DMA,
    ],
    compiler_params=pltpu.CompilerParams(use_tc_tiling_on_sc=True,
                                         needs_layout_passes=False),
)
def _gather(table_hbm, tail_hbm, idx_hbm, out_hbm,
            idx_buf, plist, lane_list, q_list,
            chunk_v, rows_v, p_stage, sem):
    wid = lax.axis_index("s") * _NC + lax.axis_index("c")
    lo = wid * _R
    off_max = NB_AL - _W - lo

    def scan_piece(pp, cnt):
        pltpu.sync_copy(idx_hbm.at[pl.ds(pp * 1024, 1024)], idx_buf)

        def group(g, cnt):
            i_vec = idx_buf[pl.ds(g * 16, 16)]
            il_vec = i_vec - lo
            p_vec = pp * 1024 + g * 16 + _iota16()
            m = (il_vec >= 0) & (il_vec < _R)
            mi = m.astype(jnp.int32)
            pos = cnt + plsc.cumsum(mi) - 1
            tgt = jnp.where(m, pos, _LDUMP)
            plsc.store_scatter(plist, [tgt],
                               jnp.where(m, (il_vec << 14) | p_vec, 0))
            return cnt + jnp.sum(mi)

        return lax.fori_loop(0, 64, group, cnt)

    cnt = lax.fori_loop(0, 16, scan_piece, jnp.int32(0))

    def off_l_of(c):
        return jnp.minimum(c * _W, off_max)

    def fire_chunk(c, par):
        off = lo + off_l_of(c)
        return [pltpu.async_copy(
            table_hbm.at[a, pl.ds(0, 8), pl.ds(off, _W)],
            chunk_v.at[par, a], sem) for a in range(8)]

    def wait_chunk():
        for a in range(8):
            pltpu.make_async_copy(
                table_hbm.at[a, pl.ds(0, 8), pl.ds(0, _W)],
                chunk_v.at[0, a], sem).wait()

    def process_chunk(piece, pc, par, off_l, width):
        def sub(gg, scnt):
            qpos = piece * _CAP + gg * 16
            v = plist[pl.ds(qpos, 16)]
            il_vec = v >> 14
            ql_vec = gg * 16 + _iota16()
            m = ((ql_vec < pc) & (il_vec >= off_l) &
                 (il_vec < off_l + width))
            mi = m.astype(jnp.int32)
            pos = scnt + plsc.cumsum(mi) - 1
            tgt = jnp.where(m, pos, _SDUMP)
            plsc.store_scatter(lane_list, [tgt], il_vec - off_l)
            plsc.store_scatter(q_list, [tgt], ql_vec)
            return scnt + jnp.sum(mi)

        scnt = lax.fori_loop(0, _CAP // 16, sub, jnp.int32(0))
        par_vec = jnp.full((16,), par, jnp.int32)

        def gath(sg, carry):
            @pl.when(sg * 16 < scnt)
            def _():
                lane_vec = lane_list[pl.ds(sg * 16, 16)]
                q_vec = q_list[pl.ds(sg * 16, 16)]
                valid = (sg * 16 + _iota16()) < scnt
                lane_vec = jnp.where(valid, lane_vec, 0)
                q_vec = jnp.where(valid, q_vec, _RDUMP)
                for a in range(8):
                    a_vec = jnp.full((16,), a, jnp.int32)
                    for b in range(8):
                        v = plsc.load_gather(
                            chunk_v,
                            [par_vec, a_vec,
                             jnp.full((16,), b, jnp.int32), lane_vec])
                        plsc.store_scatter(
                            rows_v,
                            [q_vec, jnp.full((16,), a * 8 + b, jnp.int32)], v)

            return carry

        lax.fori_loop(0, _CAP // 16, gath, 0)

    def piece_body(piece, carry):
        pc = jnp.clip(cnt - piece * _CAP, 0, _CAP)

        @pl.when(pc > 0)
        def _():
            fire_chunk(0, 0)

            def chunk_body(c, carry2):
                par = c % 2
                wait_chunk()

                @pl.when(c + 1 < _NCH)
                def _fire_next():
                    fire_chunk(c + 1, 1 - par)

                process_chunk(piece, pc, par, off_l_of(c), _W)
                return carry2

            lax.fori_loop(0, _NCH, chunk_body, 0)

            cps = [pltpu.async_copy(
                tail_hbm.at[a],
                chunk_v.at[0, a, pl.ds(0, 8), pl.ds(0, 128)],
                sem) for a in range(8)]
            for cp in cps:
                cp.wait()
            process_chunk(piece, pc, 0, NB_AL - lo, 128)

            def out_body(ob, carry3):
                @pl.when(ob * 128 < pc)
                def _o():
                    for gg8 in range(8):
                        qpos = ob * 128 + gg8 * 16
                        pv = plist[pl.ds(piece * _CAP + qpos, 16)] & 0x3FFF
                        valid = (qpos + _iota16()) < pc
                        dummy = BATCH + gg8 * 16 + _iota16()
                        p_stage[0, pl.ds(gg8 * 16, 16)] = jnp.where(
                            valid, pv, dummy)
                    pltpu.async_copy(rows_v.at[pl.ds(ob * 128, 128)],
                                     out_hbm.at[p_stage.at[0]], sem).wait()

                return carry3

            lax.fori_loop(0, _NOB, out_body, 0)

        return carry

    lax.fori_loop(0, _NPIECE, piece_body, 0)


def kernel(node_ids, table):
    idx = node_ids.astype(jnp.int32)
    table_t3 = table.T.reshape(8, 8, NB)
    tail = jnp.pad(table_t3[:, :, NB_AL:], ((0, 0), (0, 0), (0, 128 - TAIL)))
    out = _gather(table_t3, tail, idx)
    return out[:BATCH, :EMB]
